# baseline (speedup 1.0000x reference)
import functools

import jax
import jax.numpy as jnp
from jax.experimental import pallas as pl

N_VIEWS = 4
D_VIEW = 128
D_FEAT = 64
N_NODES = 4
D_ATT = 100
THRESH = 0.1


def _fused_kernel(views_ref, w_enc_ref, b_enc_ref, w_dec_ref, b_dec_ref,
                  wq_ref, wk_ref, w_gnn1_ref, w_assign1_ref, w_emb_ref,
                  w_gnn2_ref, w_assign2_ref,
                  xhat_ref, xp_ref, ap_ref, ain_ref, s_ref, s2_ref, ar_ref):
    f32 = jnp.float32

    def mm(a, b):
        return jnp.dot(a, b, preferred_element_type=f32)

    def mmT(w, x):
        return jax.lax.dot_general(w, x, (((0,), (0,)), ((), ())),
                                   preferred_element_type=f32)

    def mmRT(a, b):
        return jax.lax.dot_general(a, b, (((1,), (1,)), ((), ())),
                                   preferred_element_type=f32)

    X = []
    for v in range(N_VIEWS):
        z = (mm(views_ref[v], w_enc_ref[v]) + b_enc_ref[v][None, :]).T
        z = jnp.maximum(z, 0.0)
        mu = jnp.mean(z, axis=0, keepdims=True)
        zc = z - mu
        var = jnp.mean(zc * zc, axis=0, keepdims=True)
        X.append(zc * jax.lax.rsqrt(var + 1e-5))

    scale = 1.0 / (D_ATT ** 0.5)
    M = mmRT(wq_ref[...], wk_ref[...]) * scale
    MT = mmRT(wk_ref[...], wq_ref[...]) * scale
    M11 = M[:D_FEAT, :D_FEAT]
    M21 = M[D_FEAT:, :D_FEAT]
    M12T = MT[D_FEAT:, :D_FEAT]
    M22T = MT[D_FEAT:, D_FEAT:]

    HT = [mm(M11, X[w]) for w in range(N_VIEWS)]
    R = [mm(M12T, X[v]) + M22T[:, v:v + 1] for v in range(N_VIEWS)]
    C = [mm(M21, X[w]) for w in range(N_VIEWS)]

    A_in = []
    A = []
    for v in range(N_VIEWS):
        sc = [jnp.sum(X[v] * HT[w], axis=0, keepdims=True)
              + R[v][w:w + 1] + C[w][v:v + 1] for w in range(N_VIEWS)]
        m = jnp.maximum(jnp.maximum(sc[0], sc[1]),
                        jnp.maximum(sc[2], sc[3]))
        e = [jnp.exp(s_ - m) for s_ in sc]
        inv = 1.0 / (e[0] + e[1] + e[2] + e[3])
        a_in = [e_ * inv for e_ in e]
        A_in.append(a_in)
        a = [jnp.where(a_ > THRESH, a_, 0.0) for a_ in a_in]
        a[v] = a[v] + 1.0
        inv2 = 1.0 / (a[0] + a[1] + a[2] + a[3])
        A.append(jnp.concatenate([a_ * inv2 for a_ in a], axis=0))

    H = []
    S = []
    for v in range(N_VIEWS):
        ax = A[v][0:1] * X[0]
        for w in range(1, N_VIEWS):
            ax = ax + A[v][w:w + 1] * X[w]
        H.append(jnp.maximum(mmT(w_gnn1_ref[...], ax), 0.0))
        L = mmT(w_assign1_ref[...], ax)
        m = jnp.max(L, axis=0, keepdims=True)
        e = jnp.exp(L - m)
        S.append(e * (1.0 / jnp.sum(e, axis=0, keepdims=True)))

    Xp = []
    T = []
    for n in range(N_NODES):
        xp = S[0][n:n + 1] * H[0]
        t = S[0][n:n + 1] * A[0]
        for v in range(1, N_VIEWS):
            xp = xp + S[v][n:n + 1] * H[v]
            t = t + S[v][n:n + 1] * A[v]
        Xp.append(xp)
        T.append(t)

    Ap = []
    for n in range(N_NODES):
        ap = T[n][0:1] * S[0]
        for w in range(1, N_VIEWS):
            ap = ap + T[n][w:w + 1] * S[w]
        Ap.append(ap)

    Xe = [jnp.maximum(mmT(w_emb_ref[...], Xp[n]), 0.0) for n in range(N_NODES)]

    H2 = []
    S2 = []
    for n in range(N_NODES):
        axe = Ap[n][0:1] * Xe[0]
        for m_ in range(1, N_NODES):
            axe = axe + Ap[n][m_:m_ + 1] * Xe[m_]
        H2.append(jnp.maximum(mmT(w_gnn2_ref[...], axe), 0.0))
        L = mmT(w_assign2_ref[...], axe)
        m = jnp.max(L, axis=0, keepdims=True)
        e = jnp.exp(L - m)
        S2.append(e * (1.0 / jnp.sum(e, axis=0, keepdims=True)))

    V = []
    for n in range(N_NODES):
        vv = Ap[n][0:1] * S2[0]
        for m_ in range(1, N_NODES):
            vv = vv + Ap[n][m_:m_ + 1] * S2[m_]
        V.append(vv)

    Ar = []
    for v in range(N_VIEWS):
        xr = S2[0][v:v + 1] * H2[0]
        ar = S2[0][v:v + 1] * V[0]
        for n in range(1, N_NODES):
            xr = xr + S2[n][v:v + 1] * H2[n]
            ar = ar + S2[n][v:v + 1] * V[n]
        Ar.append(ar)
        xhat_ref[v] = mm(xr.T, w_dec_ref[v]) + b_dec_ref[v][None, :]

    for n in range(N_NODES):
        xp_ref[:, n * D_FEAT:(n + 1) * D_FEAT] = Xp[n].T
    ap_ref[...] = jnp.concatenate(Ap, axis=0).T
    ain_ref[...] = jnp.concatenate(sum(A_in, []), axis=0).T
    s_ref[...] = jnp.concatenate(S, axis=0).T
    s2_ref[...] = jnp.concatenate(S2, axis=0).T
    ar_ref[...] = jnp.concatenate(Ar, axis=0).T


@functools.partial(jax.jit, static_argnames=("tb", "interpret"))
def _run(views, W_enc, b_enc, W_dec, b_dec, Wq, Wk, W_gnn1, W_assign1,
         W_emb, W_gnn2, W_assign2, tb=2048, interpret=False):
    batch = views.shape[1]
    grid = (batch // tb,)
    args = (views, W_enc, b_enc, W_dec, b_dec, Wq, Wk, W_gnn1, W_assign1,
            W_emb, W_gnn2, W_assign2)

    def wspec(x):
        nd = x.ndim
        return pl.BlockSpec(x.shape, lambda i: (0,) * nd)

    in_specs = [pl.BlockSpec((N_VIEWS, tb, D_VIEW), lambda i: (0, i, 0))]
    in_specs += [wspec(a) for a in args[1:]]
    out_specs = [
        pl.BlockSpec((N_VIEWS, tb, D_VIEW), lambda i: (0, i, 0)),
        pl.BlockSpec((tb, N_NODES * D_FEAT), lambda i: (i, 0)),
        pl.BlockSpec((tb, N_NODES * N_NODES), lambda i: (i, 0)),
        pl.BlockSpec((tb, N_VIEWS * N_VIEWS), lambda i: (i, 0)),
        pl.BlockSpec((tb, N_VIEWS * N_NODES), lambda i: (i, 0)),
        pl.BlockSpec((tb, N_NODES * N_VIEWS), lambda i: (i, 0)),
        pl.BlockSpec((tb, N_VIEWS * N_VIEWS), lambda i: (i, 0)),
    ]
    f32 = jnp.float32
    out_shape = [
        jax.ShapeDtypeStruct((N_VIEWS, batch, D_VIEW), f32),
        jax.ShapeDtypeStruct((batch, N_NODES * D_FEAT), f32),
        jax.ShapeDtypeStruct((batch, N_NODES * N_NODES), f32),
        jax.ShapeDtypeStruct((batch, N_VIEWS * N_VIEWS), f32),
        jax.ShapeDtypeStruct((batch, N_VIEWS * N_NODES), f32),
        jax.ShapeDtypeStruct((batch, N_NODES * N_VIEWS), f32),
        jax.ShapeDtypeStruct((batch, N_VIEWS * N_VIEWS), f32),
    ]
    x_hat, xp, ap, a_in, s, s2, ar = pl.pallas_call(
        _fused_kernel,
        grid=grid,
        in_specs=in_specs,
        out_specs=out_specs,
        out_shape=out_shape,
        interpret=interpret,
    )(*args)
    return (x_hat,
            xp.reshape(batch, N_NODES, D_FEAT),
            ap.reshape(batch, N_NODES, N_NODES),
            a_in.reshape(batch, N_VIEWS, N_VIEWS),
            s.reshape(batch, N_VIEWS, N_NODES),
            s2.reshape(batch, N_NODES, N_VIEWS),
            ar.reshape(batch, N_VIEWS, N_VIEWS))


def kernel(views, W_enc, b_enc, W_dec, b_dec, Wq, Wk, W_gnn1, W_assign1,
           W_emb, W_gnn2, W_assign2):
    return _run(views, W_enc, b_enc, W_dec, b_dec, Wq, Wk, W_gnn1,
                W_assign1, W_emb, W_gnn2, W_assign2)


# speedup vs baseline: 1.0087x; 1.0087x over previous
import functools

import jax
import jax.numpy as jnp
from jax.experimental import pallas as pl

N_VIEWS = 4
D_VIEW = 128
D_FEAT = 64
N_NODES = 4
D_ATT = 100
THRESH = 0.1


def _fused_kernel(views_ref, w_enc_ref, b_enc_ref, w_dec_ref, b_dec_ref,
                  wq_ref, wk_ref, w_gnn1_ref, w_assign1_ref, w_emb_ref,
                  w_gnn2_ref, w_assign2_ref,
                  xhat_ref, xp_ref, ap_ref, ain_ref, s_ref, s2_ref, ar_ref):
    f32 = jnp.float32

    def mm(a, b):
        return jnp.dot(a, b, preferred_element_type=f32)

    def mmT(w, x):
        return jax.lax.dot_general(w, x, (((0,), (0,)), ((), ())),
                                   preferred_element_type=f32)

    def mmRT(a, b):
        return jax.lax.dot_general(a, b, (((1,), (1,)), ((), ())),
                                   preferred_element_type=f32)

    X = []
    for v in range(N_VIEWS):
        z = (mm(views_ref[v], w_enc_ref[v]) + b_enc_ref[v][None, :]).T
        z = jnp.maximum(z, 0.0)
        mu = jnp.mean(z, axis=0, keepdims=True)
        zc = z - mu
        var = jnp.mean(zc * zc, axis=0, keepdims=True)
        X.append(zc * jax.lax.rsqrt(var + 1e-5))

    scale = 1.0 / (D_ATT ** 0.5)
    M = mmRT(wq_ref[...], wk_ref[...]) * scale
    MT = mmRT(wk_ref[...], wq_ref[...]) * scale
    M11 = M[:D_FEAT, :D_FEAT]
    M21 = M[D_FEAT:, :D_FEAT]
    M12T = MT[D_FEAT:, :D_FEAT]
    M22T = MT[D_FEAT:, D_FEAT:]

    HT = [mm(M11, X[w]) for w in range(N_VIEWS)]
    R = [mm(M12T, X[v]) + M22T[:, v:v + 1] for v in range(N_VIEWS)]
    C = [mm(M21, X[w]) for w in range(N_VIEWS)]

    A_in = []
    A = []
    for v in range(N_VIEWS):
        sc = [jnp.sum(X[v] * HT[w], axis=0, keepdims=True)
              + R[v][w:w + 1] + C[w][v:v + 1] for w in range(N_VIEWS)]
        m = jnp.maximum(jnp.maximum(sc[0], sc[1]),
                        jnp.maximum(sc[2], sc[3]))
        e = [jnp.exp(s_ - m) for s_ in sc]
        inv = 1.0 / (e[0] + e[1] + e[2] + e[3])
        a_in = [e_ * inv for e_ in e]
        A_in.append(a_in)
        a = [jnp.where(a_ > THRESH, a_, 0.0) for a_ in a_in]
        a[v] = a[v] + 1.0
        inv2 = 1.0 / (a[0] + a[1] + a[2] + a[3])
        A.append(jnp.concatenate([a_ * inv2 for a_ in a], axis=0))

    H = []
    S = []
    for v in range(N_VIEWS):
        ax = A[v][0:1] * X[0]
        for w in range(1, N_VIEWS):
            ax = ax + A[v][w:w + 1] * X[w]
        H.append(jnp.maximum(mmT(w_gnn1_ref[...], ax), 0.0))
        L = mmT(w_assign1_ref[...], ax)
        m = jnp.max(L, axis=0, keepdims=True)
        e = jnp.exp(L - m)
        S.append(e * (1.0 / jnp.sum(e, axis=0, keepdims=True)))

    Xp = []
    T = []
    for n in range(N_NODES):
        xp = S[0][n:n + 1] * H[0]
        t = S[0][n:n + 1] * A[0]
        for v in range(1, N_VIEWS):
            xp = xp + S[v][n:n + 1] * H[v]
            t = t + S[v][n:n + 1] * A[v]
        Xp.append(xp)
        T.append(t)

    Ap = []
    for n in range(N_NODES):
        ap = T[n][0:1] * S[0]
        for w in range(1, N_VIEWS):
            ap = ap + T[n][w:w + 1] * S[w]
        Ap.append(ap)

    Xe = [jnp.maximum(mmT(w_emb_ref[...], Xp[n]), 0.0) for n in range(N_NODES)]

    H2 = []
    S2 = []
    for n in range(N_NODES):
        axe = Ap[n][0:1] * Xe[0]
        for m_ in range(1, N_NODES):
            axe = axe + Ap[n][m_:m_ + 1] * Xe[m_]
        H2.append(jnp.maximum(mmT(w_gnn2_ref[...], axe), 0.0))
        L = mmT(w_assign2_ref[...], axe)
        m = jnp.max(L, axis=0, keepdims=True)
        e = jnp.exp(L - m)
        S2.append(e * (1.0 / jnp.sum(e, axis=0, keepdims=True)))

    V = []
    for n in range(N_NODES):
        vv = Ap[n][0:1] * S2[0]
        for m_ in range(1, N_NODES):
            vv = vv + Ap[n][m_:m_ + 1] * S2[m_]
        V.append(vv)

    Ar = []
    for v in range(N_VIEWS):
        xr = S2[0][v:v + 1] * H2[0]
        ar = S2[0][v:v + 1] * V[0]
        for n in range(1, N_NODES):
            xr = xr + S2[n][v:v + 1] * H2[n]
            ar = ar + S2[n][v:v + 1] * V[n]
        Ar.append(ar)
        xhat_ref[v] = mm(xr.T, w_dec_ref[v]) + b_dec_ref[v][None, :]

    for n in range(N_NODES):
        xp_ref[:, n * D_FEAT:(n + 1) * D_FEAT] = Xp[n].T
    ap_ref[...] = jnp.concatenate(Ap, axis=0).T
    ain_ref[...] = jnp.concatenate(sum(A_in, []), axis=0).T
    s_ref[...] = jnp.concatenate(S, axis=0).T
    s2_ref[...] = jnp.concatenate(S2, axis=0).T
    ar_ref[...] = jnp.concatenate(Ar, axis=0).T


@functools.partial(jax.jit, static_argnames=("tb", "interpret"))
def _run(views, W_enc, b_enc, W_dec, b_dec, Wq, Wk, W_gnn1, W_assign1,
         W_emb, W_gnn2, W_assign2, tb=1024, interpret=False):
    batch = views.shape[1]
    grid = (batch // tb,)
    args = (views, W_enc, b_enc, W_dec, b_dec, Wq, Wk, W_gnn1, W_assign1,
            W_emb, W_gnn2, W_assign2)

    def wspec(x):
        nd = x.ndim
        return pl.BlockSpec(x.shape, lambda i: (0,) * nd)

    in_specs = [pl.BlockSpec((N_VIEWS, tb, D_VIEW), lambda i: (0, i, 0))]
    in_specs += [wspec(a) for a in args[1:]]
    out_specs = [
        pl.BlockSpec((N_VIEWS, tb, D_VIEW), lambda i: (0, i, 0)),
        pl.BlockSpec((tb, N_NODES * D_FEAT), lambda i: (i, 0)),
        pl.BlockSpec((tb, N_NODES * N_NODES), lambda i: (i, 0)),
        pl.BlockSpec((tb, N_VIEWS * N_VIEWS), lambda i: (i, 0)),
        pl.BlockSpec((tb, N_VIEWS * N_NODES), lambda i: (i, 0)),
        pl.BlockSpec((tb, N_NODES * N_VIEWS), lambda i: (i, 0)),
        pl.BlockSpec((tb, N_VIEWS * N_VIEWS), lambda i: (i, 0)),
    ]
    f32 = jnp.float32
    out_shape = [
        jax.ShapeDtypeStruct((N_VIEWS, batch, D_VIEW), f32),
        jax.ShapeDtypeStruct((batch, N_NODES * D_FEAT), f32),
        jax.ShapeDtypeStruct((batch, N_NODES * N_NODES), f32),
        jax.ShapeDtypeStruct((batch, N_VIEWS * N_VIEWS), f32),
        jax.ShapeDtypeStruct((batch, N_VIEWS * N_NODES), f32),
        jax.ShapeDtypeStruct((batch, N_NODES * N_VIEWS), f32),
        jax.ShapeDtypeStruct((batch, N_VIEWS * N_VIEWS), f32),
    ]
    x_hat, xp, ap, a_in, s, s2, ar = pl.pallas_call(
        _fused_kernel,
        grid=grid,
        in_specs=in_specs,
        out_specs=out_specs,
        out_shape=out_shape,
        interpret=interpret,
    )(*args)
    return (x_hat,
            xp.reshape(batch, N_NODES, D_FEAT),
            ap.reshape(batch, N_NODES, N_NODES),
            a_in.reshape(batch, N_VIEWS, N_VIEWS),
            s.reshape(batch, N_VIEWS, N_NODES),
            s2.reshape(batch, N_NODES, N_VIEWS),
            ar.reshape(batch, N_VIEWS, N_VIEWS))


def kernel(views, W_enc, b_enc, W_dec, b_dec, Wq, Wk, W_gnn1, W_assign1,
           W_emb, W_gnn2, W_assign2):
    return _run(views, W_enc, b_enc, W_dec, b_dec, Wq, Wk, W_gnn1,
                W_assign1, W_emb, W_gnn2, W_assign2)
